# baseline (speedup 1.0000x reference)
import jax
import jax.numpy as jnp
from jax import lax
from jax.experimental import pallas as pl
from jax.experimental.pallas import tpu as pltpu
from jax.experimental.pallas import tpu_sc as plsc

_N = 50000
_E = 1600000
_HID = 32
_NPAD = 50048
_NSC = 2
_NTILES = 16
_NW = _NSC * _NTILES
_CH = 400
_NCH = _E // (_NW * _CH)
_WB = _NPAD // _NTILES
_BR = 5000
_GRID = _N // _BR


def _make_scatter():
  mesh = plsc.VectorSubcoreMesh(core_axis_name="c", subcore_axis_name="s")
  out_type = jax.ShapeDtypeStruct((_NSC, _NPAD, _HID), jnp.float32)
  scratch = [
      pltpu.VMEM((2, _CH), jnp.int32),
      pltpu.VMEM((3, _CH), jnp.int32),
      pltpu.VMEM((2, _CH, _HID), jnp.float32),
      pltpu.VMEM_SHARED((_NPAD, _HID), jnp.float32),
      pltpu.SemaphoreType.DMA,
      pltpu.SemaphoreType.DMA,
      pltpu.SemaphoreType.DMA,
  ]

  def body(ei, xin, z2, acc_out, src_v, dst_v, rows_v, acc_sh,
           isem, gsem, ssem):
    c = lax.axis_index("c")
    t = lax.axis_index("s")
    wid = c * _NTILES + t
    pltpu.sync_copy(z2.at[pl.ds(t * _WB, _WB)], acc_sh.at[pl.ds(t * _WB, _WB)])
    plsc.subcore_barrier()

    def idx_dma(ch, ssl, dsl):
      e0 = ch * _CH
      pltpu.async_copy(ei.at[0, pl.ds(e0, _CH)], src_v.at[ssl], isem)
      pltpu.async_copy(ei.at[1, pl.ds(e0, _CH)], dst_v.at[dsl], isem)

    def idx_wait(ssl, dsl):
      pltpu.make_async_copy(ei.at[0, pl.ds(0, _CH)], src_v.at[ssl], isem).wait()
      pltpu.make_async_copy(ei.at[1, pl.ds(0, _CH)], dst_v.at[dsl], isem).wait()

    def gather(ssl, rsl):
      pltpu.async_copy(xin.at[src_v.at[ssl]], rows_v.at[rsl], gsem)

    def gather_wait(ssl, rsl):
      pltpu.make_async_copy(xin.at[src_v.at[ssl]], rows_v.at[rsl], gsem).wait()

    def scat(rsl, dsl):
      pltpu.async_copy(rows_v.at[rsl], acc_sh.at[dst_v.at[dsl]], ssem,
                       add=True)

    def scat_wait(rsl, dsl):
      pltpu.make_async_copy(rows_v.at[rsl], acc_sh.at[dst_v.at[dsl]],
                            ssem).wait()

    def cid(i):
      return i * _NW + wid

    idx_dma(cid(0), 0, 0)
    idx_wait(0, 0)
    gather(0, 0)
    idx_dma(cid(1), 1, 1)

    def loop(i, carry):
      a = lax.rem(i, 2)
      b = lax.rem(i + 1, 2)
      d = lax.rem(i, 3)
      dn = lax.rem(i + 1, 3)
      dp = lax.rem(i + 2, 3)
      gather_wait(a, a)

      @pl.when(i >= 1)
      def _():
        scat_wait(b, dp)

      scat(a, d)
      idx_wait(b, dn)
      gather(b, b)

      @pl.when(i + 2 <= _NCH - 1)
      def _():
        idx_dma(cid(i + 2), a, dp)

      return carry

    lax.fori_loop(0, _NCH - 1, loop, 0)
    al = (_NCH - 1) % 2
    dl = (_NCH - 1) % 3
    bl = (_NCH - 2) % 2
    dlp = (_NCH - 2) % 3
    gather_wait(al, al)
    scat_wait(bl, dlp)
    scat(al, dl)
    scat_wait(al, dl)

    plsc.subcore_barrier()
    pltpu.sync_copy(acc_sh.at[pl.ds(t * _WB, _WB)],
                    acc_out.at[c, pl.ds(t * _WB, _WB)])

  return pl.kernel(
      body, out_type=out_type, mesh=mesh, scratch_types=scratch,
      compiler_params=pltpu.CompilerParams(use_tc_tiling_on_sc=False))


def _make_s_scatter():
  mesh = plsc.VectorSubcoreMesh(core_axis_name="c", subcore_axis_name="s")
  out_type = jax.ShapeDtypeStruct((_NSC, _NPAD, 16), jnp.float32)
  scratch = [
      pltpu.VMEM((3, _CH), jnp.int32),
      pltpu.VMEM((2, _CH), jnp.float32),
      pltpu.VMEM((2, _CH, 16), jnp.float32),
      pltpu.VMEM_SHARED((_NPAD, 16), jnp.float32),
      pltpu.SemaphoreType.DMA,
      pltpu.SemaphoreType.DMA,
  ]

  def body(ei, ea1, z16, s_out, dst_v, ea_v, stage, s_sh, isem, ssem):
    c = lax.axis_index("c")
    t = lax.axis_index("s")
    wid = c * _NTILES + t
    pltpu.sync_copy(z16.at[pl.ds(t * _WB, _WB)], s_sh.at[pl.ds(t * _WB, _WB)])
    zv = jnp.zeros((16,), jnp.float32)

    def zrow(i, carry):
      stage[lax.div(i, _CH), lax.rem(i, _CH), :] = zv
      return carry

    lax.fori_loop(0, 2 * _CH, zrow, 0)
    plsc.subcore_barrier()

    col0 = jnp.zeros((16,), jnp.int32)
    lane = lax.iota(jnp.int32, 16)

    def idx_dma(ch, esl, dsl):
      e0 = ch * _CH
      pltpu.async_copy(ei.at[1, pl.ds(e0, _CH)], dst_v.at[dsl], isem)
      pltpu.async_copy(ea1.at[pl.ds(e0, _CH)], ea_v.at[esl], isem)

    def idx_wait(esl, dsl):
      pltpu.make_async_copy(ei.at[1, pl.ds(0, _CH)], dst_v.at[dsl],
                            isem).wait()
      pltpu.make_async_copy(ea1.at[pl.ds(0, _CH)], ea_v.at[esl], isem).wait()

    def assemble(esl):
      for k in range(_CH // 16):
        vals = ea_v[esl, pl.ds(k * 16, 16)]
        plsc.store_scatter(stage.at[esl], [lane + (k * 16), col0], vals)

    def scat(esl, dsl):
      pltpu.async_copy(stage.at[esl], s_sh.at[dst_v.at[dsl]], ssem, add=True)

    def scat_wait(esl, dsl):
      pltpu.make_async_copy(stage.at[esl], s_sh.at[dst_v.at[dsl]],
                            ssem).wait()

    def cid(i):
      return i * _NW + wid

    idx_dma(cid(0), 0, 0)
    idx_wait(0, 0)
    assemble(0)
    idx_dma(cid(1), 1, 1)

    def loop(i, carry):
      a = lax.rem(i, 2)
      b = lax.rem(i + 1, 2)
      d = lax.rem(i, 3)
      dn = lax.rem(i + 1, 3)
      dp = lax.rem(i + 2, 3)

      @pl.when(i >= 1)
      def _():
        scat_wait(b, dp)

      scat(a, d)
      idx_wait(b, dn)
      assemble(b)

      @pl.when(i + 2 <= _NCH - 1)
      def _():
        idx_dma(cid(i + 2), a, dp)

      return carry

    lax.fori_loop(0, _NCH - 1, loop, 0)
    al = (_NCH - 1) % 2
    dl = (_NCH - 1) % 3
    bl = (_NCH - 2) % 2
    dlp = (_NCH - 2) % 3
    scat_wait(bl, dlp)
    scat(al, dl)
    scat_wait(al, dl)
    plsc.subcore_barrier()
    pltpu.sync_copy(s_sh.at[pl.ds(t * _WB, _WB)],
                    s_out.at[c, pl.ds(t * _WB, _WB)])

  return pl.kernel(
      body, out_type=out_type, mesh=mesh, scratch_types=scratch,
      compiler_params=pltpu.CompilerParams(
          use_tc_tiling_on_sc=False, needs_layout_passes=False))


def _pre_body(x_ref, w1_ref, b1_ref, wi1_ref, o_ref):
  h = jnp.dot(x_ref[...], w1_ref[...], preferred_element_type=jnp.float32)
  h = jnp.maximum(h + b1_ref[...], 0.0)
  o_ref[...] = jnp.dot(h, wi1_ref[...], preferred_element_type=jnp.float32)


def _comb1_body(p_ref, xin_ref, s_ref, wenc_ref, bias_ref, wi2_ref, o_ref):
  sv = s_ref[0, :, 0] + s_ref[1, :, 0]
  h = (p_ref[0] + p_ref[1] + xin_ref[...]
       + sv[:, None] * wenc_ref[...] + bias_ref[...])
  h = jnp.maximum(h, 0.0)
  o_ref[...] = jnp.dot(h, wi2_ref[...], preferred_element_type=jnp.float32)


def _comb2_body(q_ref, xin_ref, s_ref, wenc_ref, bias_ref, wh_ref, bh_ref,
                o_ref, acc_ref):
  i = pl.program_id(0)

  @pl.when(i == 0)
  def _():
    acc_ref[...] = jnp.zeros_like(acc_ref)

  sv = s_ref[0, :, 0] + s_ref[1, :, 0]
  h = (q_ref[0] + q_ref[1] + xin_ref[...]
       + sv[:, None] * wenc_ref[...] + bias_ref[...])
  h = jnp.maximum(h, 0.0)
  acc_ref[...] += jnp.sum(h, axis=0, keepdims=True)

  @pl.when(i == pl.num_programs(0) - 1)
  def _():
    pooled = acc_ref[...] * (1.0 / _N)
    o_ref[...] = (jnp.sum(pooled * wh_ref[...], axis=1, keepdims=True)
                  + bh_ref[...])


def _full_spec(shape):
  return pl.BlockSpec(shape, lambda i: tuple(0 for _ in shape))


def kernel(x, edge_index, edge_attr, bn_mean, bn_var, bn_gamma, bn_beta,
           W1, b1, Win1, Wenc1, bias1, Win2, Wenc2, bias2, Wh, bh):
  f32 = jnp.float32
  a = bn_gamma * lax.rsqrt(bn_var + 1e-5)
  cvec = bn_beta - bn_mean * a
  w1p = (W1 * a[None, :]).T
  b1p = (b1 + W1 @ cvec)[None, :]
  wi1t = Win1.T
  wi2t = Win2.T
  wenc1r = Wenc1.T
  wenc2r = Wenc2.T
  bias1r = bias1[None, :]
  bias2r = bias2[None, :]
  bhr = bh[None, :]

  ea1 = edge_attr.reshape(_E)
  z2 = jnp.zeros((_NPAD, _HID), f32)
  z16 = jnp.zeros((_NPAD, 16), f32)

  xin1 = pl.pallas_call(
      _pre_body,
      grid=(_GRID,),
      in_specs=[
          pl.BlockSpec((_BR, 25), lambda i: (i, 0)),
          _full_spec((25, _HID)),
          _full_spec((1, _HID)),
          _full_spec((_HID, _HID)),
      ],
      out_specs=pl.BlockSpec((_BR, _HID), lambda i: (i, 0)),
      out_shape=jax.ShapeDtypeStruct((_NPAD, _HID), f32),
  )(x, w1p, b1p, wi1t)

  s16 = _make_s_scatter()(edge_index, ea1, z16)
  s16 = s16[0] if isinstance(s16, (list, tuple)) else s16
  s_acc = s16[:, :, 0:1]

  z2d = z2 + 0.0 * s_acc[0, 0, 0]
  res1 = _make_scatter()(edge_index, xin1, z2d)
  acc1 = res1[0] if isinstance(res1, (list, tuple)) else res1

  xin2 = pl.pallas_call(
      _comb1_body,
      grid=(_GRID,),
      in_specs=[
          pl.BlockSpec((_NSC, _BR, _HID), lambda i: (0, i, 0)),
          pl.BlockSpec((_BR, _HID), lambda i: (i, 0)),
          pl.BlockSpec((_NSC, _BR, 1), lambda i: (0, i, 0)),
          _full_spec((1, _HID)),
          _full_spec((1, _HID)),
          _full_spec((_HID, _HID)),
      ],
      out_specs=pl.BlockSpec((_BR, _HID), lambda i: (i, 0)),
      out_shape=jax.ShapeDtypeStruct((_NPAD, _HID), f32),
  )(acc1, xin1, s_acc, wenc1r, bias1r, wi2t)

  res2 = _make_scatter()(edge_index, xin2, z2)
  acc2 = res2[0] if isinstance(res2, (list, tuple)) else res2

  out = pl.pallas_call(
      _comb2_body,
      grid=(_GRID,),
      in_specs=[
          pl.BlockSpec((_NSC, _BR, _HID), lambda i: (0, i, 0)),
          pl.BlockSpec((_BR, _HID), lambda i: (i, 0)),
          pl.BlockSpec((_NSC, _BR, 1), lambda i: (0, i, 0)),
          _full_spec((1, _HID)),
          _full_spec((1, _HID)),
          _full_spec((1, _HID)),
          _full_spec((1, 1)),
      ],
      out_specs=_full_spec((1, 1)),
      out_shape=jax.ShapeDtypeStruct((1, 1), f32),
      scratch_shapes=[pltpu.VMEM((1, _HID), f32)],
  )(acc2, xin2, s_acc, wenc2r, bias2r, Wh, bhr)
  return out


# speedup vs baseline: 1.0614x; 1.0614x over previous
import jax
import jax.numpy as jnp
from jax import lax
from jax.experimental import pallas as pl
from jax.experimental.pallas import tpu as pltpu
from jax.experimental.pallas import tpu_sc as plsc

_N = 50000
_E = 1600000
_HID = 32
_NPAD = 50048
_NSC = 2
_NTILES = 16
_NW = _NSC * _NTILES
_CH = 400
_NCH = _E // (_NW * _CH)
_WB = _NPAD // _NTILES
_BR = 5000
_GRID = _N // _BR


def _make_scatter():
  mesh = plsc.VectorSubcoreMesh(core_axis_name="c", subcore_axis_name="s")
  out_type = jax.ShapeDtypeStruct((_NSC, _NPAD, _HID), jnp.float32)
  scratch = [
      pltpu.VMEM((2, _CH), jnp.int32),
      pltpu.VMEM((3, _CH), jnp.int32),
      pltpu.VMEM((2, _CH, _HID), jnp.float32),
      pltpu.VMEM_SHARED((_NPAD, _HID), jnp.float32),
      pltpu.SemaphoreType.DMA,
      pltpu.SemaphoreType.DMA,
      pltpu.SemaphoreType.DMA,
  ]

  def body(ei, xin, z2, acc_out, src_v, dst_v, rows_v, acc_sh,
           isem, gsem, ssem):
    c = lax.axis_index("c")
    t = lax.axis_index("s")
    wid = c * _NTILES + t
    pltpu.sync_copy(z2.at[pl.ds(t * _WB, _WB)], acc_sh.at[pl.ds(t * _WB, _WB)])
    plsc.subcore_barrier()

    def idx_dma(ch, ssl, dsl):
      e0 = ch * _CH
      pltpu.async_copy(ei.at[0, pl.ds(e0, _CH)], src_v.at[ssl], isem)
      pltpu.async_copy(ei.at[1, pl.ds(e0, _CH)], dst_v.at[dsl], isem)

    def idx_wait(ssl, dsl):
      pltpu.make_async_copy(ei.at[0, pl.ds(0, _CH)], src_v.at[ssl], isem).wait()
      pltpu.make_async_copy(ei.at[1, pl.ds(0, _CH)], dst_v.at[dsl], isem).wait()

    def gather(ssl, rsl):
      pltpu.async_copy(xin.at[src_v.at[ssl]], rows_v.at[rsl], gsem)

    def gather_wait(ssl, rsl):
      pltpu.make_async_copy(xin.at[src_v.at[ssl]], rows_v.at[rsl], gsem).wait()

    def scat(rsl, dsl):
      pltpu.async_copy(rows_v.at[rsl], acc_sh.at[dst_v.at[dsl]], ssem,
                       add=True)

    def scat_wait(rsl, dsl):
      pltpu.make_async_copy(rows_v.at[rsl], acc_sh.at[dst_v.at[dsl]],
                            ssem).wait()

    def cid(i):
      return i * _NW + wid

    idx_dma(cid(0), 0, 0)
    idx_wait(0, 0)
    gather(0, 0)
    idx_dma(cid(1), 1, 1)

    def loop(i, carry):
      a = lax.rem(i, 2)
      b = lax.rem(i + 1, 2)
      d = lax.rem(i, 3)
      dn = lax.rem(i + 1, 3)
      dp = lax.rem(i + 2, 3)
      gather_wait(a, a)

      @pl.when(i >= 1)
      def _():
        scat_wait(b, dp)

      scat(a, d)
      idx_wait(b, dn)
      gather(b, b)

      @pl.when(i + 2 <= _NCH - 1)
      def _():
        idx_dma(cid(i + 2), a, dp)

      return carry

    lax.fori_loop(0, _NCH - 1, loop, 0)
    al = (_NCH - 1) % 2
    dl = (_NCH - 1) % 3
    bl = (_NCH - 2) % 2
    dlp = (_NCH - 2) % 3
    gather_wait(al, al)
    scat_wait(bl, dlp)
    scat(al, dl)
    scat_wait(al, dl)

    plsc.subcore_barrier()
    pltpu.sync_copy(acc_sh.at[pl.ds(t * _WB, _WB)],
                    acc_out.at[c, pl.ds(t * _WB, _WB)])

  return pl.kernel(
      body, out_type=out_type, mesh=mesh, scratch_types=scratch,
      compiler_params=pltpu.CompilerParams(use_tc_tiling_on_sc=False))


def _make_s_scatter():
  mesh = plsc.VectorSubcoreMesh(core_axis_name="c", subcore_axis_name="s")
  out_type = jax.ShapeDtypeStruct((_NSC, _NPAD, 16), jnp.float32)
  scratch = [
      pltpu.VMEM((3, _CH), jnp.int32),
      pltpu.VMEM((2, _CH), jnp.float32),
      pltpu.VMEM((2, _CH, 16), jnp.float32),
      pltpu.VMEM_SHARED((_NPAD, 16), jnp.float32),
      pltpu.SemaphoreType.DMA,
      pltpu.SemaphoreType.DMA,
  ]

  def body(ei, ea1, z16, s_out, dst_v, ea_v, stage, s_sh, isem, ssem):
    c = lax.axis_index("c")
    t = lax.axis_index("s")
    wid = c * _NTILES + t
    pltpu.sync_copy(z16.at[pl.ds(t * _WB, _WB)], s_sh.at[pl.ds(t * _WB, _WB)])
    zv = jnp.zeros((16,), jnp.float32)

    def zrow(i, carry):
      stage[lax.div(i, _CH), lax.rem(i, _CH), :] = zv
      return carry

    lax.fori_loop(0, 2 * _CH, zrow, 0)
    plsc.subcore_barrier()

    col0 = jnp.zeros((16,), jnp.int32)
    lane = lax.iota(jnp.int32, 16)

    def idx_dma(ch, esl, dsl):
      e0 = ch * _CH
      pltpu.async_copy(ei.at[1, pl.ds(e0, _CH)], dst_v.at[dsl], isem)
      pltpu.async_copy(ea1.at[pl.ds(e0, _CH)], ea_v.at[esl], isem)

    def idx_wait(esl, dsl):
      pltpu.make_async_copy(ei.at[1, pl.ds(0, _CH)], dst_v.at[dsl],
                            isem).wait()
      pltpu.make_async_copy(ea1.at[pl.ds(0, _CH)], ea_v.at[esl], isem).wait()

    def assemble(esl):
      for k in range(_CH // 16):
        vals = ea_v[esl, pl.ds(k * 16, 16)]
        plsc.store_scatter(stage.at[esl], [lane + (k * 16), col0], vals)

    def scat(esl, dsl):
      pltpu.async_copy(stage.at[esl], s_sh.at[dst_v.at[dsl]], ssem, add=True)

    def scat_wait(esl, dsl):
      pltpu.make_async_copy(stage.at[esl], s_sh.at[dst_v.at[dsl]],
                            ssem).wait()

    def cid(i):
      return i * _NW + wid

    idx_dma(cid(0), 0, 0)
    idx_wait(0, 0)
    assemble(0)
    idx_dma(cid(1), 1, 1)

    def loop(i, carry):
      a = lax.rem(i, 2)
      b = lax.rem(i + 1, 2)
      d = lax.rem(i, 3)
      dn = lax.rem(i + 1, 3)
      dp = lax.rem(i + 2, 3)

      @pl.when(i >= 1)
      def _():
        scat_wait(b, dp)

      scat(a, d)
      idx_wait(b, dn)
      assemble(b)

      @pl.when(i + 2 <= _NCH - 1)
      def _():
        idx_dma(cid(i + 2), a, dp)

      return carry

    lax.fori_loop(0, _NCH - 1, loop, 0)
    al = (_NCH - 1) % 2
    dl = (_NCH - 1) % 3
    bl = (_NCH - 2) % 2
    dlp = (_NCH - 2) % 3
    scat_wait(bl, dlp)
    scat(al, dl)
    scat_wait(al, dl)
    plsc.subcore_barrier()
    pltpu.sync_copy(s_sh.at[pl.ds(t * _WB, _WB)],
                    s_out.at[c, pl.ds(t * _WB, _WB)])

  return pl.kernel(
      body, out_type=out_type, mesh=mesh, scratch_types=scratch,
      compiler_params=pltpu.CompilerParams(
          use_tc_tiling_on_sc=False, needs_layout_passes=False))


def _pre_body(x_ref, w1_ref, b1_ref, wi1_ref, o_ref):
  h = jnp.dot(x_ref[...], w1_ref[...], preferred_element_type=jnp.float32)
  h = jnp.maximum(h + b1_ref[...], 0.0)
  o_ref[...] = jnp.dot(h, wi1_ref[...], preferred_element_type=jnp.float32)


def _comb1_body(p_ref, xin_ref, s_ref, wenc_ref, bias_ref, wi2_ref, o_ref):
  sv = s_ref[0, :, 0] + s_ref[1, :, 0]
  h = (p_ref[0] + p_ref[1] + xin_ref[...]
       + sv[:, None] * wenc_ref[...] + bias_ref[...])
  h = jnp.maximum(h, 0.0)
  o_ref[...] = jnp.dot(h, wi2_ref[...], preferred_element_type=jnp.float32)


def _comb2_body(q_ref, xin_ref, s_ref, wenc_ref, bias_ref, wh_ref, bh_ref,
                o_ref, acc_ref):
  i = pl.program_id(0)

  @pl.when(i == 0)
  def _():
    acc_ref[...] = jnp.zeros_like(acc_ref)

  sv = s_ref[0, :, 0] + s_ref[1, :, 0]
  h = (q_ref[0] + q_ref[1] + xin_ref[...]
       + sv[:, None] * wenc_ref[...] + bias_ref[...])
  h = jnp.maximum(h, 0.0)
  acc_ref[...] += jnp.sum(h, axis=0, keepdims=True)

  @pl.when(i == pl.num_programs(0) - 1)
  def _():
    pooled = acc_ref[...] * (1.0 / _N)
    o_ref[...] = (jnp.sum(pooled * wh_ref[...], axis=1, keepdims=True)
                  + bh_ref[...])


def _full_spec(shape):
  return pl.BlockSpec(shape, lambda i: tuple(0 for _ in shape))


def kernel(x, edge_index, edge_attr, bn_mean, bn_var, bn_gamma, bn_beta,
           W1, b1, Win1, Wenc1, bias1, Win2, Wenc2, bias2, Wh, bh):
  f32 = jnp.float32
  a = bn_gamma * lax.rsqrt(bn_var + 1e-5)
  cvec = bn_beta - bn_mean * a
  w1p = (W1 * a[None, :]).T
  b1p = (b1 + W1 @ cvec)[None, :]
  wi1t = Win1.T
  wi2t = Win2.T
  wenc1r = Wenc1.T
  wenc2r = Wenc2.T
  bias1r = bias1[None, :]
  bias2r = bias2[None, :]
  bhr = bh[None, :]

  ea1 = edge_attr.reshape(_E)
  z2 = jnp.zeros((_NPAD, _HID), f32)
  z16 = jnp.zeros((_NPAD, 16), f32)

  xin1 = pl.pallas_call(
      _pre_body,
      grid=(_GRID,),
      in_specs=[
          pl.BlockSpec((_BR, 25), lambda i: (i, 0)),
          _full_spec((25, _HID)),
          _full_spec((1, _HID)),
          _full_spec((_HID, _HID)),
      ],
      out_specs=pl.BlockSpec((_BR, _HID), lambda i: (i, 0)),
      out_shape=jax.ShapeDtypeStruct((_NPAD, _HID), f32),
  )(x, w1p, b1p, wi1t)

  s16 = _make_s_scatter()(edge_index, ea1, z16)
  s16 = s16[0] if isinstance(s16, (list, tuple)) else s16
  s_acc = s16[:, :, 0:1]

  res1 = _make_scatter()(edge_index, xin1, z2)
  acc1 = res1[0] if isinstance(res1, (list, tuple)) else res1

  xin2 = pl.pallas_call(
      _comb1_body,
      grid=(_GRID,),
      in_specs=[
          pl.BlockSpec((_NSC, _BR, _HID), lambda i: (0, i, 0)),
          pl.BlockSpec((_BR, _HID), lambda i: (i, 0)),
          pl.BlockSpec((_NSC, _BR, 1), lambda i: (0, i, 0)),
          _full_spec((1, _HID)),
          _full_spec((1, _HID)),
          _full_spec((_HID, _HID)),
      ],
      out_specs=pl.BlockSpec((_BR, _HID), lambda i: (i, 0)),
      out_shape=jax.ShapeDtypeStruct((_NPAD, _HID), f32),
  )(acc1, xin1, s_acc, wenc1r, bias1r, wi2t)

  res2 = _make_scatter()(edge_index, xin2, z2)
  acc2 = res2[0] if isinstance(res2, (list, tuple)) else res2

  out = pl.pallas_call(
      _comb2_body,
      grid=(_GRID,),
      in_specs=[
          pl.BlockSpec((_NSC, _BR, _HID), lambda i: (0, i, 0)),
          pl.BlockSpec((_BR, _HID), lambda i: (i, 0)),
          pl.BlockSpec((_NSC, _BR, 1), lambda i: (0, i, 0)),
          _full_spec((1, _HID)),
          _full_spec((1, _HID)),
          _full_spec((1, _HID)),
          _full_spec((1, 1)),
      ],
      out_specs=_full_spec((1, 1)),
      out_shape=jax.ShapeDtypeStruct((1, 1), f32),
      scratch_shapes=[pltpu.VMEM((1, _HID), f32)],
  )(acc2, xin2, s_acc, wenc2r, bias2r, Wh, bhr)
  return out
